# baseline (speedup 1.0000x reference)
import functools

import jax
import jax.numpy as jnp
from jax import lax
from jax.experimental import pallas as pl
from jax.experimental.pallas import tpu as pltpu
from jax.experimental.pallas import tpu_sc as plsc


def _gather_kernel(seq, batch, d, dpad, n_workers, nc):
    blk = batch // n_workers
    mesh = plsc.VectorSubcoreMesh(core_axis_name="c", subcore_axis_name="s")

    @functools.partial(
        pl.kernel,
        mesh=mesh,
        compiler_params=pltpu.CompilerParams(
            use_tc_tiling_on_sc=False, needs_layout_passes=False
        ),
        out_type=jax.ShapeDtypeStruct((seq, d, batch), jnp.float32),
        scratch_types=[
            pltpu.VMEM((blk,), jnp.int32),
            pltpu.VMEM((blk, dpad), jnp.float32),
            pltpu.VMEM((d, blk), jnp.float32),
            pltpu.SemaphoreType.DMA,
        ],
    )
    def k(table_hbm, xt_hbm, out_hbm, idx_v, buf, trans, gsem):
        wid = lax.axis_index("s") * nc + lax.axis_index("c")
        b0 = wid * blk
        lane = lax.iota(jnp.int32, 16)

        def s_body(s, carry):
            pltpu.sync_copy(xt_hbm.at[s, pl.ds(b0, blk)], idx_v)
            pltpu.async_copy(table_hbm.at[idx_v], buf, gsem).wait()

            def c_body(c, inner):
                cols = jnp.full((16,), 0, jnp.int32) + c
                for kk in range(blk // 16):
                    rows = lane + (16 * kk)
                    vals = plsc.load_gather(buf, [rows, cols])
                    trans[c, pl.ds(16 * kk, 16)] = vals
                return inner

            lax.fori_loop(0, d, c_body, 0)
            pltpu.sync_copy(trans, out_hbm.at[s, :, pl.ds(b0, blk)])
            return carry

        lax.fori_loop(0, seq, s_body, 0)

    return k


def kernel(x, table):
    b, s = x.shape
    v, d = table.shape

    info = plsc.get_sparse_core_info()
    nc, ns = info.num_cores, info.num_subcores
    n_workers = nc * ns

    dpad = 128
    table_w = jnp.pad(table, ((0, 0), (0, dpad - d)))
    xt = x.T.astype(jnp.int32)
    out_t = _gather_kernel(s, b, d, dpad, n_workers, nc)(table_w, xt)
    return jnp.transpose(out_t, (2, 0, 1))


# speedup vs baseline: 1.3691x; 1.3691x over previous
import functools

import jax
import jax.numpy as jnp
from jax import lax
from jax.experimental import pallas as pl
from jax.experimental.pallas import tpu as pltpu
from jax.experimental.pallas import tpu_sc as plsc


def _gather_kernel(seq, batch, d, dpad, n_workers, nc):
    blk = batch // n_workers
    assert seq % 2 == 0
    mesh = plsc.VectorSubcoreMesh(core_axis_name="c", subcore_axis_name="s")

    @functools.partial(
        pl.kernel,
        mesh=mesh,
        compiler_params=pltpu.CompilerParams(needs_layout_passes=False),
        out_type=jax.ShapeDtypeStruct((seq, d, batch), jnp.float32),
        scratch_types=[
            pltpu.VMEM((seq, blk), jnp.int32),
            pltpu.VMEM((blk, dpad), jnp.float32),
            pltpu.VMEM((blk, dpad), jnp.float32),
            pltpu.VMEM((d, blk), jnp.float32),
            pltpu.VMEM((d, blk), jnp.float32),
            pltpu.SemaphoreType.DMA,
            pltpu.SemaphoreType.DMA,
            pltpu.SemaphoreType.DMA,
            pltpu.SemaphoreType.DMA,
        ],
    )
    def k(table_hbm, xt_hbm, out_hbm, idx_all, buf0, buf1, tr0, tr1,
          g0, g1, s0, s1):
        wid = lax.axis_index("s") * nc + lax.axis_index("c")
        b0 = wid * blk
        lane = lax.iota(jnp.int32, 16)

        pltpu.sync_copy(xt_hbm.at[:, pl.ds(b0, blk)], idx_all)

        def gather(s, buf, sem):
            pltpu.make_async_copy(table_hbm.at[idx_all.at[s]], buf, sem).start()

        def wait_gather(buf, sem):
            pltpu.make_async_copy(table_hbm.at[idx_all.at[0]], buf, sem).wait()

        def store(s, tr, sem):
            pltpu.make_async_copy(
                tr, out_hbm.at[s, :, pl.ds(b0, blk)], sem
            ).start()

        def wait_store(tr, sem):
            pltpu.make_async_copy(
                tr, out_hbm.at[0, :, pl.ds(b0, blk)], sem
            ).wait()

        def transpose(buf, tr):
            def c_body(c, inner):
                cols = jnp.full((16,), 0, jnp.int32) + c
                for kk in range(blk // 16):
                    rows = lane + (16 * kk)
                    tr[c, pl.ds(16 * kk, 16)] = plsc.load_gather(
                        buf, [rows, cols]
                    )
                return inner

            lax.fori_loop(0, d, c_body, 0)

        gather(0, buf0, g0)

        def body(j, carry):
            sa = 2 * j
            sb = 2 * j + 1

            gather(sb, buf1, g1)
            wait_gather(buf0, g0)

            @pl.when(j > 0)
            def _():
                wait_store(tr0, s0)

            transpose(buf0, tr0)
            store(sa, tr0, s0)

            @pl.when(j < seq // 2 - 1)
            def _():
                gather(sa + 2, buf0, g0)

            wait_gather(buf1, g1)

            @pl.when(j > 0)
            def _():
                wait_store(tr1, s1)

            transpose(buf1, tr1)
            store(sb, tr1, s1)
            return carry

        lax.fori_loop(0, seq // 2, body, 0)
        wait_store(tr0, s0)
        wait_store(tr1, s1)

    return k


def kernel(x, table):
    b, s = x.shape
    v, d = table.shape

    info = plsc.get_sparse_core_info()
    nc, ns = info.num_cores, info.num_subcores
    n_workers = nc * ns

    dpad = 128
    table_w = jnp.concatenate(
        [table, jnp.zeros((v, dpad - d), jnp.float32)], axis=1
    )
    xt = x.T.astype(jnp.int32)
    out_t = _gather_kernel(s, b, d, dpad, n_workers, nc)(table_w, xt)
    return jnp.transpose(out_t, (2, 0, 1))


# speedup vs baseline: 1.9339x; 1.4125x over previous
import functools

import jax
import jax.numpy as jnp
from jax import lax
from jax.experimental import pallas as pl
from jax.experimental.pallas import tpu as pltpu
from jax.experimental.pallas import tpu_sc as plsc


def _gather_kernel(seq, batch, d, dpad, n_workers, nc):
    blk = batch // n_workers
    cs = 2
    n_steps = seq // cs
    assert n_steps % 2 == 0
    mesh = plsc.VectorSubcoreMesh(core_axis_name="c", subcore_axis_name="s")

    @functools.partial(
        pl.kernel,
        mesh=mesh,
        compiler_params=pltpu.CompilerParams(needs_layout_passes=False),
        out_type=jax.ShapeDtypeStruct((seq, d, batch), jnp.float32),
        scratch_types=[
            pltpu.VMEM((cs, blk), jnp.int32),
            pltpu.VMEM((cs, blk), jnp.int32),
            pltpu.VMEM((cs * blk, dpad), jnp.float32),
            pltpu.VMEM((cs * blk, dpad), jnp.float32),
            pltpu.VMEM((cs, d, blk), jnp.float32),
            pltpu.VMEM((cs, d, blk), jnp.float32),
            pltpu.SemaphoreType.DMA,
            pltpu.SemaphoreType.DMA,
            pltpu.SemaphoreType.DMA,
            pltpu.SemaphoreType.DMA,
            pltpu.SemaphoreType.DMA,
            pltpu.SemaphoreType.DMA,
        ],
    )
    def k(table_hbm, xt_hbm, out_hbm, ix0, ix1, buf0, buf1, tr0, tr1,
          i0, i1, g0, g1, s0, s1):
        wid = lax.axis_index("s") * nc + lax.axis_index("c")
        b0 = wid * blk
        lane = lax.iota(jnp.int32, 16)

        def wait_gather(buf, gsem):
            for u in range(cs):
                pltpu.make_async_copy(
                    table_hbm.at[ix0.at[u]],
                    buf.at[pl.ds(u * blk, blk)],
                    gsem,
                ).wait()

        def store(t, tr, sem):
            pltpu.make_async_copy(
                tr, out_hbm.at[pl.ds(t * cs, cs), :, pl.ds(b0, blk)], sem
            ).start()

        def wait_store(tr, sem):
            pltpu.make_async_copy(
                tr, out_hbm.at[pl.ds(0, cs), :, pl.ds(b0, blk)], sem
            ).wait()

        def transpose(buf, tr):
            @plsc.parallel_loop(0, d, 1, unroll=2)
            def _(c):
                cols = jnp.full((16,), 0, jnp.int32) + c
                for u in range(cs):
                    for kk in range(blk // 16):
                        rows = lane + (u * blk + 16 * kk)
                        tr[u, c, pl.ds(16 * kk, 16)] = plsc.load_gather(
                            buf, [rows, cols]
                        )

        def start_fetch(t, ix, isem, buf, gsem):
            pltpu.make_async_copy(
                xt_hbm.at[pl.ds(t * cs, cs), pl.ds(b0, blk)], ix, isem
            ).start()

        def finish_fetch(ix, isem, buf, gsem):
            pltpu.make_async_copy(
                xt_hbm.at[pl.ds(0, cs), pl.ds(b0, blk)], ix, isem
            ).wait()
            for u in range(cs):
                pltpu.make_async_copy(
                    table_hbm.at[ix.at[u]],
                    buf.at[pl.ds(u * blk, blk)],
                    gsem,
                ).start()

        start_fetch(0, ix0, i0, buf0, g0)
        finish_fetch(ix0, i0, buf0, g0)
        start_fetch(1, ix1, i1, buf1, g1)

        def body(j, carry):
            ta = 2 * j
            tb = 2 * j + 1

            finish_fetch(ix1, i1, buf1, g1)

            wait_gather(buf0, g0)

            @pl.when(j > 0)
            def _():
                wait_store(tr0, s0)

            transpose(buf0, tr0)
            store(ta, tr0, s0)

            @pl.when(j < n_steps // 2 - 1)
            def _():
                start_fetch(ta + 2, ix0, i0, buf0, g0)
                finish_fetch(ix0, i0, buf0, g0)

            wait_gather(buf1, g1)

            @pl.when(j > 0)
            def _():
                wait_store(tr1, s1)

            transpose(buf1, tr1)
            store(tb, tr1, s1)

            @pl.when(j < n_steps // 2 - 1)
            def _():
                start_fetch(tb + 2, ix1, i1, buf1, g1)

            return carry

        lax.fori_loop(0, n_steps // 2, body, 0)
        wait_store(tr0, s0)
        wait_store(tr1, s1)

    return k


def kernel(x, table):
    b, s = x.shape
    v, d = table.shape

    info = plsc.get_sparse_core_info()
    nc, ns = info.num_cores, info.num_subcores
    n_workers = nc * ns

    dpad = 128
    filler = jnp.broadcast_to(table[:1, :], (v, dpad - d))
    table_w = jnp.concatenate([table, filler], axis=1)
    xt = x.T.astype(jnp.int32)
    out_t = _gather_kernel(s, b, d, dpad, n_workers, nc)(table_w, xt)
    return jnp.transpose(out_t, (2, 0, 1))
